# baseline (speedup 1.0000x reference)
import jax
import jax.numpy as jnp
from jax import lax
from jax.experimental import pallas as pl
from jax.experimental.pallas import tpu as pltpu
from jax.experimental.pallas import tpu_sc as plsc

N_NODE = 10000
N_EDGE = 320000
N_PRED = 16
N_TRAJ = 50

NP = 10240
W = 8
NTILES = 16
NS = NP // NTILES
EPT = N_EDGE // NTILES
CHUNK = 1000
NCHUNKS = EPT // CHUNK
NVA = 2
MAX_STEPS = 10
NFL = NS * W // 16


def _body(obs, snd, rcv, starts, targets, td, wout, *refs):
  (q_sh, s_sh0, s_sh1, deg_sh, sidx, wfull,
   p_t, inv_t, q_t, s_t, z_t, z1, d_t, tr_t,
   stb, tgb, ib16) = refs[:17]
  k = 17
  vas = list(refs[k:k + NVA]); k += NVA
  rb = list(refs[k:k + NCHUNKS]); k += NCHUNKS
  sem_g, sem_s, sem_z = refs[k], refs[k + 1], refs[k + 2]; k += 3
  gsems = list(refs[k:k + NVA]); k += NVA
  ssems = list(refs[k:k + NVA]); k += NVA
  cid = lax.axis_index("c")
  sid = lax.axis_index("s")
  jbase = cid * W
  lo = sid * NS
  elo = sid * EPT
  f32 = jnp.float32
  i32 = jnp.int32

  iota = lax.iota(i32, 16)
  one16 = jnp.full((16,), 1, i32)
  zero16 = jnp.full((16,), 0, i32)
  hi8 = jnp.where(iota >= 8, one16, zero16)
  cols8 = lax.rem(iota, 8)

  pltpu.sync_copy(snd.at[pl.ds(elo, EPT)], sidx)
  for c in range(NCHUNKS):
    pltpu.sync_copy(rcv.at[pl.ds(elo + c * CHUNK, CHUNK)], rb[c])
  pltpu.sync_copy(starts, stb)
  pltpu.sync_copy(targets, tgb)

  zf16 = jnp.zeros((16,), f32)
  def zero_loop(i, c):
    plsc.store_scatter(z_t, [2 * i + hi8, cols8], zf16)
    return c
  lax.fori_loop(0, NFL, zero_loop, 0)

  def zero1_loop(i, c):
    z1[pl.ds(i * 16, 16)] = zf16
    return c
  lax.fori_loop(0, NS // 16, zero1_loop, 0)

  of16 = jnp.ones((16,), f32)
  def ones_loop(i, c):
    wfull[pl.ds(i * 16, 16)] = of16
    return c
  lax.fori_loop(0, EPT // 16, ones_loop, 0)

  pltpu.sync_copy(z1, deg_sh.at[pl.ds(lo, NS)])
  plsc.subcore_barrier()
  pltpu.async_copy(wfull, deg_sh.at[sidx], sem_s, add=True).wait()
  plsc.subcore_barrier()

  pltpu.sync_copy(deg_sh.at[pl.ds(lo, NS)], d_t)
  def inv_loop(m, c):
    dv = d_t[pl.ds(m * 16, 16)]
    rv = 1.0 / dv
    d_t[pl.ds(m * 16, 16)] = rv
    for j in range(8):
      r0 = rv[2 * j]
      r1 = rv[2 * j + 1]
      inv_t[m * 8 + j, :] = jnp.where(
          iota < 8, jnp.full((16,), r0, f32), jnp.full((16,), r1, f32))
    return c
  lax.fori_loop(0, NS // 16, inv_loop, 0)
  pltpu.sync_copy(d_t, deg_sh.at[pl.ds(lo, NS)])
  plsc.subcore_barrier()

  pltpu.async_copy(deg_sh.at[sidx], wfull, sem_g).wait()
  descs = []
  for j in range(W):
    descs.append(pltpu.async_copy(
        wfull, wout.at[jbase + j, pl.ds(elo, EPT)], sem_s))
  for d in descs:
    d.wait()

  lane_pred = jbase + cols8
  st_pair = plsc.load_gather(stb, [lane_pred])
  tg_pair = plsc.load_gather(tgb, [lane_pred])
  steps16 = jnp.maximum(5 * (tg_pair - st_pair), 0)

  ib16[...] = st_pair * NTILES + sid
  pltpu.async_copy(obs.at[ib16], tr_t, sem_g).wait()
  def trin_loop(i, c):
    pv = plsc.load_gather(tr_t, [cols8, 2 * i + hi8])
    p_t[i, :] = pv
    qv = pv * inv_t[i, :]
    plsc.store_scatter(q_t, [2 * i + hi8, cols8], qv)
    return c
  lax.fori_loop(0, NFL, trin_loop, 0)
  pltpu.sync_copy(q_t, q_sh.at[pl.ds(lo, NS)])

  pltpu.sync_copy(z_t, s_sh0.at[pl.ds(lo, NS)])
  plsc.subcore_barrier()

  def one_step(k, scur, snxt):
    zd = pltpu.async_copy(z_t, snxt.at[pl.ds(lo, NS)], sem_z)
    gd = {}
    def gissue(c):
      return pltpu.async_copy(q_sh.at[sidx.at[pl.ds(c * CHUNK, CHUNK)]],
                              vas[c % NVA], gsems[c % NVA])
    for c in range(min(NVA, NCHUNKS)):
      gd[c] = gissue(c)
    for c in range(NCHUNKS):
      gd[c].wait()
      sd = pltpu.async_copy(vas[c % NVA], scur.at[rb[c]],
                            ssems[c % NVA], add=True)
      if c + NVA < NCHUNKS:
        sd.wait()
        gd[c + NVA] = gissue(c + NVA)
      elif c >= NCHUNKS - NVA:
        sd.wait()
    zd.wait()
    plsc.subcore_barrier()
    pltpu.sync_copy(scur.at[pl.ds(lo, NS)], s_t)
    mask = steps16 > k
    def bloop(i, c):
      sv = plsc.load_gather(s_t, [2 * i + hi8, cols8])
      pv = jnp.where(mask, sv, p_t[i, :])
      p_t[i, :] = pv
      qv = pv * inv_t[i, :]
      plsc.store_scatter(q_t, [2 * i + hi8, cols8], qv)
      return c
    lax.fori_loop(0, NFL, bloop, 0)
    pltpu.sync_copy(q_t, q_sh.at[pl.ds(lo, NS)])
    plsc.subcore_barrier()

  def dstep(m, carry):
    one_step(2 * m, s_sh0, s_sh1)
    one_step(2 * m + 1, s_sh1, s_sh0)
    return carry
  lax.fori_loop(0, MAX_STEPS // 2, dstep, 0)

  def trout_loop(i, c):
    plsc.store_scatter(tr_t, [cols8, 2 * i + hi8], p_t[i, :])
    return c
  lax.fori_loop(0, NFL, trout_loop, 0)
  descs = []
  for j in range(W):
    descs.append(pltpu.async_copy(
        tr_t.at[j], td.at[jbase + j, pl.ds(lo, NS)], sem_s))
  for d in descs:
    d.wait()


def _scratch_types():
  f32 = jnp.float32
  i32 = jnp.int32
  types = [
      pltpu.VMEM_SHARED((NP, W), f32),
      pltpu.VMEM_SHARED((NP, W), f32),
      pltpu.VMEM_SHARED((NP, W), f32),
      pltpu.VMEM_SHARED((NP,), f32),
      pltpu.VMEM((EPT,), i32),
      pltpu.VMEM((EPT,), f32),
      pltpu.VMEM((NFL, 16), f32),
      pltpu.VMEM((NFL, 16), f32),
      pltpu.VMEM((NS, W), f32),
      pltpu.VMEM((NS, W), f32),
      pltpu.VMEM((NS, W), f32),
      pltpu.VMEM((NS,), f32),
      pltpu.VMEM((NS,), f32),
      pltpu.VMEM((N_PRED, NS), f32),
      pltpu.VMEM((N_PRED,), i32),
      pltpu.VMEM((N_PRED,), i32),
      pltpu.VMEM((N_PRED,), i32),
  ]
  types += [pltpu.VMEM((CHUNK, W), f32) for _ in range(NVA)]
  types += [pltpu.VMEM((CHUNK,), i32) for _ in range(NCHUNKS)]
  types += [pltpu.SemaphoreType.DMA] * 3
  types += [pltpu.SemaphoreType.DMA for _ in range(2 * NVA)]
  return types


@jax.jit
def kernel(observations, edge_index, observed, starts, targets,
           pairwise_node_features):
  del observed, pairwise_node_features
  obs_p = jnp.pad(observations, ((0, 0), (0, NP - N_NODE)))
  obs_p = obs_p.reshape(N_TRAJ * NTILES, NS)
  mesh = plsc.VectorSubcoreMesh(core_axis_name="c", subcore_axis_name="s",
                                num_cores=2, num_subcores=NTILES)
  f32 = jnp.float32
  td_p, wout = pl.kernel(
      _body,
      out_type=[
          jax.ShapeDtypeStruct((N_PRED, NP), f32),
          jax.ShapeDtypeStruct((N_PRED, N_EDGE), f32),
      ],
      mesh=mesh,
      compiler_params=pltpu.CompilerParams(use_tc_tiling_on_sc=False,
                                           needs_layout_passes=False),
      scratch_types=_scratch_types(),
  )(obs_p, edge_index[0], edge_index[1], starts, targets)
  return td_p[:, :N_NODE], wout


# speedup vs baseline: 1.0351x; 1.0351x over previous
import jax
import jax.numpy as jnp
from jax import lax
from jax.experimental import pallas as pl
from jax.experimental.pallas import tpu as pltpu
from jax.experimental.pallas import tpu_sc as plsc

N_NODE = 10000
N_EDGE = 320000
N_PRED = 16
N_TRAJ = 50

NP = 10240
W = 8
NTILES = 16
NS = NP // NTILES
EPT = N_EDGE // NTILES
CHUNK = 1000
NCHUNKS = EPT // CHUNK
NVA = 2
MAX_STEPS = 10
NFL = NS * W // 16


def _body(obs, snd, rcv, starts, targets, td, wout, *refs):
  (q_sh, s_sh, deg_sh, sidx, wfull,
   p_t, inv_t, q_t, s_t, z_t, z1, d_t, tr_t,
   stb, tgb, ib16) = refs[:16]
  k = 16
  vas = list(refs[k:k + NVA]); k += NVA
  rb = list(refs[k:k + NCHUNKS]); k += NCHUNKS
  sem_g, sem_s, sem_o, sem_w = refs[k:k + 4]; k += 4
  gsems = list(refs[k:k + NVA]); k += NVA
  ssems = list(refs[k:k + NVA]); k += NVA
  cid = lax.axis_index("c")
  sid = lax.axis_index("s")
  jbase = cid * W
  lo = sid * NS
  elo = sid * EPT
  f32 = jnp.float32
  i32 = jnp.int32

  iota = lax.iota(i32, 16)
  one16 = jnp.full((16,), 1, i32)
  zero16 = jnp.full((16,), 0, i32)
  hi8 = jnp.where(iota >= 8, one16, zero16)
  cols8 = lax.rem(iota, 8)

  pltpu.sync_copy(snd.at[pl.ds(elo, EPT)], sidx)
  for c in range(NCHUNKS):
    pltpu.sync_copy(rcv.at[pl.ds(elo + c * CHUNK, CHUNK)], rb[c])
  pltpu.sync_copy(starts, stb)
  pltpu.sync_copy(targets, tgb)

  lane_pred = jbase + cols8
  st_pair = plsc.load_gather(stb, [lane_pred])
  tg_pair = plsc.load_gather(tgb, [lane_pred])
  steps16 = jnp.maximum(5 * (tg_pair - st_pair), 0)

  ib16[...] = st_pair * NTILES + sid
  od = pltpu.async_copy(obs.at[ib16], tr_t, sem_o)

  zf16 = jnp.zeros((16,), f32)
  def zero_loop(i, c):
    plsc.store_scatter(z_t, [2 * i + hi8, cols8], zf16)
    return c
  lax.fori_loop(0, NFL, zero_loop, 0)

  def zero1_loop(i, c):
    z1[pl.ds(i * 16, 16)] = zf16
    return c
  lax.fori_loop(0, NS // 16, zero1_loop, 0)

  of16 = jnp.ones((16,), f32)
  def ones_loop(i, c):
    wfull[pl.ds(i * 16, 16)] = of16
    return c
  lax.fori_loop(0, EPT // 16, ones_loop, 0)

  pltpu.sync_copy(z1, deg_sh.at[pl.ds(lo, NS)])
  plsc.subcore_barrier()
  pltpu.async_copy(wfull, deg_sh.at[sidx], sem_s, add=True).wait()
  plsc.subcore_barrier()

  pltpu.sync_copy(deg_sh.at[pl.ds(lo, NS)], d_t)
  def inv_loop(m, c):
    dv = d_t[pl.ds(m * 16, 16)]
    rv = 1.0 / dv
    d_t[pl.ds(m * 16, 16)] = rv
    for j in range(8):
      r0 = rv[2 * j]
      r1 = rv[2 * j + 1]
      inv_t[m * 8 + j, :] = jnp.where(
          iota < 8, jnp.full((16,), r0, f32), jnp.full((16,), r1, f32))
    return c
  lax.fori_loop(0, NS // 16, inv_loop, 0)
  pltpu.sync_copy(d_t, deg_sh.at[pl.ds(lo, NS)])
  plsc.subcore_barrier()

  pltpu.async_copy(deg_sh.at[sidx], wfull, sem_g).wait()
  wdescs = []
  for j in range(W):
    wdescs.append(pltpu.async_copy(
        wfull, wout.at[jbase + j, pl.ds(elo, EPT)], sem_w))

  od.wait()
  def trin_loop(i, c):
    pv = plsc.load_gather(tr_t, [cols8, 2 * i + hi8])
    p_t[i, :] = pv
    qv = pv * inv_t[i, :]
    plsc.store_scatter(q_t, [2 * i + hi8, cols8], qv)
    return c
  lax.fori_loop(0, NFL, trin_loop, 0)

  def step(k, carry):
    d1 = pltpu.async_copy(q_t, q_sh.at[pl.ds(lo, NS)], sem_g)
    d2 = pltpu.async_copy(z_t, s_sh.at[pl.ds(lo, NS)], sem_o)
    d1.wait()
    d2.wait()
    plsc.subcore_barrier()
    gd = {}
    def gissue(c):
      return pltpu.async_copy(q_sh.at[sidx.at[pl.ds(c * CHUNK, CHUNK)]],
                              vas[c % NVA], gsems[c % NVA])
    for c in range(min(NVA, NCHUNKS)):
      gd[c] = gissue(c)
    for c in range(NCHUNKS):
      gd[c].wait()
      sd = pltpu.async_copy(vas[c % NVA], s_sh.at[rb[c]],
                            ssems[c % NVA], add=True)
      if c + NVA < NCHUNKS:
        sd.wait()
        gd[c + NVA] = gissue(c + NVA)
      elif c >= NCHUNKS - NVA:
        sd.wait()
    plsc.subcore_barrier()
    pltpu.sync_copy(s_sh.at[pl.ds(lo, NS)], s_t)
    mask = steps16 > k
    def bloop(i, c):
      sv = plsc.load_gather(s_t, [2 * i + hi8, cols8])
      pv = jnp.where(mask, sv, p_t[i, :])
      p_t[i, :] = pv
      qv = pv * inv_t[i, :]
      plsc.store_scatter(q_t, [2 * i + hi8, cols8], qv)
      return c
    lax.fori_loop(0, NFL, bloop, 0)
    return carry
  lax.fori_loop(0, MAX_STEPS, step, 0)

  def trout_loop(i, c):
    plsc.store_scatter(tr_t, [cols8, 2 * i + hi8], p_t[i, :])
    return c
  lax.fori_loop(0, NFL, trout_loop, 0)
  descs = []
  for j in range(W):
    descs.append(pltpu.async_copy(
        tr_t.at[j], td.at[jbase + j, pl.ds(lo, NS)], sem_s))
  for d in descs:
    d.wait()
  for d in wdescs:
    d.wait()


def _scratch_types():
  f32 = jnp.float32
  i32 = jnp.int32
  types = [
      pltpu.VMEM_SHARED((NP, W), f32),
      pltpu.VMEM_SHARED((NP, W), f32),
      pltpu.VMEM_SHARED((NP,), f32),
      pltpu.VMEM((EPT,), i32),
      pltpu.VMEM((EPT,), f32),
      pltpu.VMEM((NFL, 16), f32),
      pltpu.VMEM((NFL, 16), f32),
      pltpu.VMEM((NS, W), f32),
      pltpu.VMEM((NS, W), f32),
      pltpu.VMEM((NS, W), f32),
      pltpu.VMEM((NS,), f32),
      pltpu.VMEM((NS,), f32),
      pltpu.VMEM((N_PRED, NS), f32),
      pltpu.VMEM((N_PRED,), i32),
      pltpu.VMEM((N_PRED,), i32),
      pltpu.VMEM((N_PRED,), i32),
  ]
  types += [pltpu.VMEM((CHUNK, W), f32) for _ in range(NVA)]
  types += [pltpu.VMEM((CHUNK,), i32) for _ in range(NCHUNKS)]
  types += [pltpu.SemaphoreType.DMA] * 4
  types += [pltpu.SemaphoreType.DMA for _ in range(2 * NVA)]
  return types


@jax.jit
def kernel(observations, edge_index, observed, starts, targets,
           pairwise_node_features):
  del observed, pairwise_node_features
  obs_p = jnp.pad(observations, ((0, 0), (0, NP - N_NODE)))
  obs_p = obs_p.reshape(N_TRAJ * NTILES, NS)
  mesh = plsc.VectorSubcoreMesh(core_axis_name="c", subcore_axis_name="s",
                                num_cores=2, num_subcores=NTILES)
  f32 = jnp.float32
  td_p, wout = pl.kernel(
      _body,
      out_type=[
          jax.ShapeDtypeStruct((N_PRED, NP), f32),
          jax.ShapeDtypeStruct((N_PRED, N_EDGE), f32),
      ],
      mesh=mesh,
      compiler_params=pltpu.CompilerParams(use_tc_tiling_on_sc=False,
                                           needs_layout_passes=False),
      scratch_types=_scratch_types(),
  )(obs_p, edge_index[0], edge_index[1], starts, targets)
  return td_p[:, :N_NODE], wout
